# baseline (speedup 1.0000x reference)
import math

import jax
import jax.numpy as jnp
from jax import lax
from jax.experimental import pallas as pl
from jax.experimental.pallas import tpu as pltpu
from jax.experimental.pallas import tpu_sc as plsc

_B, _S, _D = 4, 2048, 1024
_SCALE = math.sqrt(_D)
_NW = 32
_PPW = _S // _NW
_RPW = _B * _PPW
_CH = 16
_NCH = _RPW // _CH
_QPB = _PPW // _CH
_LANES = 16
_VPR = _D // _LANES


def _embed_kernel(x_hbm, tok_hbm, pos_hbm, out_hbm, idx_v, pos_v,
                  tok0, tok1, gs0, gs1, ss0, ss1, psem):
    toks = (tok0, tok1)
    gsems = (gs0, gs1)
    ssems = (ss0, ss1)

    c = lax.axis_index("c")
    s = lax.axis_index("s")
    wid = s * 2 + c
    pos0 = wid * _PPW

    pload = pltpu.async_copy(pos_hbm.at[pl.ds(pos0, _PPW)], pos_v, psem)

    for b in range(_B):
        pltpu.sync_copy(x_hbm.at[pl.ds(b * _S + pos0, _PPW)],
                        idx_v.at[pl.ds(b * _PPW, _PPW)])

    def start_gather(ch):
        return pltpu.async_copy(
            tok_hbm.at[idx_v.at[pl.ds(ch * _CH, _CH)]],
            toks[ch % 2], gsems[ch % 2])

    loads = [None] * _NCH
    stores = [None] * _NCH
    loads[0] = start_gather(0)
    pload.wait()
    for ch in range(_NCH):
        b, q = ch // _QPB, ch % _QPB
        buf = ch % 2
        if ch + 1 < _NCH:
            if ch >= 1 and stores[ch - 1] is not None:
                stores[ch - 1].wait()
            loads[ch + 1] = start_gather(ch + 1)
        loads[ch].wait()

        q16 = q * _CH

        def row_body(r, carry):
            for k in range(_VPR):
                t = toks[buf][r, pl.ds(k * _LANES, _LANES)]
                pv = pos_v[q16 + r, pl.ds(k * _LANES, _LANES)]
                toks[buf][r, pl.ds(k * _LANES, _LANES)] = t * _SCALE + pv
            return carry
        lax.fori_loop(0, _CH, row_body, 0, unroll=False)

        out_base = b * _S + pos0 + q16
        stores[ch] = pltpu.async_copy(
            toks[buf], out_hbm.at[pl.ds(out_base, _CH)], ssems[buf])
    stores[_NCH - 2].wait()
    stores[_NCH - 1].wait()


def kernel(x, token_table, pos_table):
    xf = x.reshape(_B * _S).astype(jnp.int32)
    mesh = plsc.VectorSubcoreMesh(core_axis_name="c", subcore_axis_name="s")
    run = pl.kernel(
        _embed_kernel,
        out_type=jax.ShapeDtypeStruct((_B * _S, _D), jnp.float32),
        mesh=mesh,
        scratch_types=[
            pltpu.VMEM((_RPW,), jnp.int32),
            pltpu.VMEM((_PPW, _D), jnp.float32),
            pltpu.VMEM((_CH, _D), jnp.float32),
            pltpu.VMEM((_CH, _D), jnp.float32),
            pltpu.SemaphoreType.DMA,
            pltpu.SemaphoreType.DMA,
            pltpu.SemaphoreType.DMA,
            pltpu.SemaphoreType.DMA,
            pltpu.SemaphoreType.DMA,
        ],
    )
    out = run(xf, token_table, pos_table)
    return out.reshape(_B, _S, _D)
